# baseline (speedup 1.0000x reference)
import functools

import jax
import jax.numpy as jnp
import numpy as np
from jax import lax
from jax.experimental import pallas as pl
from jax.experimental.pallas import tpu as pltpu
from jax.experimental.pallas import tpu_sc as plsc

N = 10000
E = 160000
D = 128
HID = 4
G = 4
FS = 32
CW = FS * HID
ROW = 80
BLK = 5
EB = ROW * BLK
NT = 16
NROWS = E // ROW
RPT = NROWS // NT
NBLK = RPT // BLK
RNODE = N // NT
NUM_GRAPHS = 64
OUT_DIM = 10
ROWBLK = 1000
NRB = N // ROWBLK

_cc = np.arange(CW)
_PERM = np.stack([(32 * g + (_cc % 32)) * 4 + (_cc // 32) for g in range(G)])
_PERM_FLAT = _PERM.reshape(-1)


@functools.cache
def _get_rel_k():
    return functools.partial(
        pl.kernel,
        out_type=jax.ShapeDtypeStruct((E * 4,), jnp.float32),
        mesh=plsc.VectorSubcoreMesh(core_axis_name="c", subcore_axis_name="s"),
        scratch_types=[
            pltpu.VMEM((BLK, ROW), jnp.int32),
            pltpu.VMEM((BLK, ROW), jnp.int32),
            pltpu.VMEM((EB, 16), jnp.float32),
            pltpu.VMEM((EB, 16), jnp.float32),
            pltpu.VMEM((EB * 4,), jnp.float32),
            pltpu.SemaphoreType.DMA,
            pltpu.SemaphoreType.DMA,
        ],
        compiler_params=pltpu.CompilerParams(
            needs_layout_passes=False, use_tc_tiling_on_sc=False),
    )(_rel_body)


_NGRP = NROWS // BLK


def _rel_body(pos_hbm, src_hbm, dst_hbm, rel_hbm, sidx, didx, ps, pd, relv,
              sem_a, sem_g):
    c = lax.axis_index("c")
    s = lax.axis_index("s")
    w = s * 2 + c
    iota = lax.iota(jnp.int32, 16)

    def body(bi, _):
        grp = w + 32 * bi

        @pl.when(grp < _NGRP)
        def _():
            rowbase = grp * BLK
            ca1 = pltpu.async_copy(src_hbm.at[pl.ds(rowbase, BLK)], sidx, sem_a)
            ca2 = pltpu.async_copy(dst_hbm.at[pl.ds(rowbase, BLK)], didx, sem_a)
            ca1.wait()
            ca2.wait()
            cps = [pltpu.async_copy(pos_hbm.at[sidx.at[u]],
                                    ps.at[pl.ds(ROW * u, ROW)], sem_g)
                   for u in range(BLK)]
            cps += [pltpu.async_copy(pos_hbm.at[didx.at[u]],
                                     pd.at[pl.ds(ROW * u, ROW)], sem_g)
                    for u in range(BLK)]
            for cp in cps:
                cp.wait()
            i1 = iota % 4
            ibase = iota // 4

            def ext(ci, _):
                i0 = 4 * ci + ibase
                pv = plsc.load_gather(ps, [i0, i1])
                dv = plsc.load_gather(pd, [i0, i1])
                u = plsc.bitcast(pv - dv, jnp.int32)
                u = (u + 0x7FFF + (lax.shift_right_logical(u, 16) & 1)) & ~0xFFFF
                relv[pl.ds(16 * ci, 16)] = plsc.bitcast(u, jnp.float32)
                return 0

            lax.fori_loop(0, EB * 4 // 16, ext, 0)
            pltpu.sync_copy(relv, rel_hbm.at[pl.ds(rowbase * (ROW * 4), EB * 4)])

        return 0

    lax.fori_loop(0, (_NGRP + 31) // 32, body, 0)


@functools.cache
def _get_msg_k():
    return functools.partial(
        pl.kernel,
        out_type=jax.ShapeDtypeStruct((G * N, CW), jnp.float32),
        mesh=plsc.VectorSubcoreMesh(core_axis_name="c", subcore_axis_name="s"),
        scratch_types=[
            pltpu.VMEM_SHARED((N, CW), jnp.float32),
            pltpu.VMEM((BLK, ROW), jnp.int32),
            pltpu.VMEM((BLK, ROW), jnp.int32),
            pltpu.VMEM((BLK, ROW), jnp.int32),
            pltpu.VMEM((EB * 4 + 16,), jnp.float32),
            pltpu.VMEM((EB, FS), jnp.float32),
            pltpu.VMEM((2, ROW, CW), jnp.float32),
            pltpu.VMEM((3, CW), jnp.float32),
            pltpu.VMEM((CW,), jnp.float32),
            pltpu.SemaphoreType.DMA,
            pltpu.SemaphoreType.DMA,
            pltpu.SemaphoreType.DMA,
            pltpu.SemaphoreType.DMA,
        ],
        compiler_params=pltpu.CompilerParams(
            needs_layout_passes=False, use_tc_tiling_on_sc=False),
    )(_msg_body)


def _msg_body(xflat_hbm, relf_hbm, src_hbm, dst_hbm, W_hbm, b_hbm, zeros_hbm,
              out_hbm, acc, sidx, didx, gidx, relv, xg, msg, Wg, bg,
              sem_a, sem_b, sem_g, sem_s):
    c = lax.axis_index("c")
    s = lax.axis_index("s")

    for q in range(2):
        g = 2 * q + c
        pltpu.sync_copy(W_hbm.at[g], Wg)
        pltpu.sync_copy(b_hbm.at[g], bg)
        pltpu.sync_copy(zeros_hbm.at[pl.ds(s * RNODE, RNODE)],
                        acc.at[pl.ds(s * RNODE, RNODE)])
        plsc.subcore_barrier()

        gN = jnp.full((16,), g * N, dtype=jnp.int32)
        wv = [(Wg[0, pl.ds(16 * j, 16)], Wg[1, pl.ds(16 * j, 16)],
               Wg[2, pl.ds(16 * j, 16)], bg[pl.ds(16 * j, 16)])
              for j in range(CW // 16)]

        def blk(bi, _):
            rowbase = s * RPT + bi * BLK
            ebase = rowbase * ROW
            ca1 = pltpu.async_copy(src_hbm.at[pl.ds(rowbase, BLK)], sidx, sem_a)
            ca2 = pltpu.async_copy(dst_hbm.at[pl.ds(rowbase, BLK)], didx, sem_a)
            cb = pltpu.async_copy(relf_hbm.at[pl.ds(ebase * 4, EB * 4)],
                                  relv.at[pl.ds(0, EB * 4)], sem_b)
            ca1.wait()
            ca2.wait()
            for u in range(BLK):
                for i in range(ROW // 16):
                    gidx[u, pl.ds(16 * i, 16)] = sidx[u, pl.ds(16 * i, 16)] + gN
            cps = [pltpu.async_copy(xflat_hbm.at[gidx.at[u]],
                                    xg.at[pl.ds(ROW * u, ROW)], sem_g)
                   for u in range(BLK)]
            for cp in cps:
                cp.wait()
            cb.wait()

            prev = [None]
            for u in range(BLK):
                mb = msg.at[u % 2]

                def edge4(eg, _, u=u, mb=mb):
                    rv = relv[pl.ds(16 * (20 * u + eg), 16)]
                    for ie in range(4):
                        e = 4 * eg + ie
                        ee = ROW * u + e
                        r0 = jnp.full((16,), rv[4 * ie], dtype=jnp.float32)
                        r1 = jnp.full((16,), rv[4 * ie + 1], dtype=jnp.float32)
                        r2 = jnp.full((16,), rv[4 * ie + 2], dtype=jnp.float32)
                        xv0 = xg[ee, pl.ds(0, 16)]
                        xv1 = xg[ee, pl.ds(16, 16)]
                        for j in range(CW // 16):
                            w0, w1, w2, bb = wv[j]
                            sp = jnp.maximum(w0 * r0 + w1 * r1 + w2 * r2 + bb, 0.0)
                            mb[e, pl.ds(16 * j, 16)] = sp * (xv0 if j % 2 == 0 else xv1)
                    return 0

                lax.fori_loop(0, ROW // 4, edge4, 0, unroll=2)
                if prev[0] is not None:
                    prev[0].wait()
                prev[0] = pltpu.async_copy(mb, acc.at[didx.at[u]], sem_s, add=True)
            prev[0].wait()
            return 0

        lax.fori_loop(0, NBLK, blk, 0)
        plsc.subcore_barrier()
        pltpu.sync_copy(acc.at[pl.ds(s * RNODE, RNODE)],
                        out_hbm.at[pl.ds(g * N + s * RNODE, RNODE)])
        plsc.subcore_barrier()


def _dense0_body(agg_ref, w_ref, b_ref, o_ref):
    h = jnp.dot(agg_ref[0], w_ref[0], preferred_element_type=jnp.float32)
    for g in range(1, G):
        h += jnp.dot(agg_ref[g], w_ref[g], preferred_element_type=jnp.float32)
    h = jnp.maximum(h + b_ref[...], 0.0)
    for g in range(G):
        o_ref[g] = h[:, 32 * g:32 * (g + 1)]


def _dense1_body(agg_ref, w_ref, b_ref, batch_ref, pool_ref, cnt_ref):
    i = pl.program_id(0)

    @pl.when(i == 0)
    def _():
        pool_ref[...] = jnp.zeros_like(pool_ref)
        cnt_ref[...] = jnp.zeros_like(cnt_ref)

    h = jnp.dot(agg_ref[0], w_ref[0], preferred_element_type=jnp.float32)
    for g in range(1, G):
        h += jnp.dot(agg_ref[g], w_ref[g], preferred_element_type=jnp.float32)
    h = jnp.maximum(h + b_ref[...], 0.0)
    bt = batch_ref[0]
    ohT = (jnp.broadcast_to(bt, (NUM_GRAPHS, ROWBLK)) ==
           lax.broadcasted_iota(jnp.int32, (NUM_GRAPHS, ROWBLK), 0)
           ).astype(jnp.float32)
    pool_ref[...] += lax.dot_general(ohT, h, (((1,), (0,)), ((), ())),
                                     preferred_element_type=jnp.float32,
                                     precision=lax.Precision.HIGHEST)
    cnt_ref[...] += lax.dot_general(ohT, jnp.ones((ROWBLK, D), jnp.float32),
                                    (((1,), (0,)), ((), ())),
                                    preferred_element_type=jnp.float32, precision=lax.Precision.HIGHEST)


def _head_body(pool_ref, cnt_ref, w_ref, b_ref, o_ref):
    mean = pool_ref[...] / jnp.maximum(cnt_ref[...], 1.0)
    logits = jnp.dot(mean, w_ref[...], preferred_element_type=jnp.float32) + b_ref[...]
    m = jnp.max(logits, axis=1, keepdims=True)
    sh = logits - m
    o_ref[...] = sh - jnp.log(jnp.sum(jnp.exp(sh), axis=1, keepdims=True))


def _prep_w(W_in, b_in, W_out):
    W_in = W_in.astype(jnp.bfloat16).astype(jnp.float32)
    Wp = W_in[:, _PERM_FLAT].reshape(3, G, CW).transpose(1, 0, 2)
    bp = b_in[_PERM_FLAT].reshape(G, CW)
    Wop = W_out[_PERM_FLAT].reshape(G, CW, D)
    return Wp, bp, Wop


def kernel(x, pos, edge_index, batch, W_in0, b_in0, W_out0, b_out0,
           W_in1, b_in1, W_out1, b_out1, fc1_W, fc1_b):
    src2 = edge_index[0].reshape(NROWS, ROW).astype(jnp.int32)
    dst2 = edge_index[1].reshape(NROWS, ROW).astype(jnp.int32)
    pos4 = jnp.pad(pos, ((0, 0), (0, 13)))
    zeros = jnp.zeros((N, CW), jnp.float32)
    batch3 = batch.reshape(NRB, 1, ROWBLK).astype(jnp.int32)

    relf = _get_rel_k()(pos4, src2, dst2)

    Wp0, bp0, Wop0 = _prep_w(W_in0, b_in0, W_out0)
    Wp1, bp1, Wop1 = _prep_w(W_in1, b_in1, W_out1)

    xflat0 = x.reshape(N, G, FS).transpose(1, 0, 2).reshape(G * N, FS)
    agg0 = _get_msg_k()(xflat0, relf, src2, dst2, Wp0, bp0, zeros)

    h1flat = pl.pallas_call(
        _dense0_body,
        grid=(NRB,),
        in_specs=[
            pl.BlockSpec((G, ROWBLK, D), lambda i: (0, i, 0)),
            pl.BlockSpec((G, D, D), lambda i: (0, 0, 0)),
            pl.BlockSpec((1, D), lambda i: (0, 0)),
        ],
        out_specs=pl.BlockSpec((G, ROWBLK, FS), lambda i: (0, i, 0)),
        out_shape=jax.ShapeDtypeStruct((G, N, FS), jnp.float32),
    )(agg0.reshape(G, N, CW), Wop0, b_out0.reshape(1, D))

    agg1 = _get_msg_k()(h1flat.reshape(G * N, FS), relf, src2, dst2, Wp1, bp1, zeros)

    pool, cnt = pl.pallas_call(
        _dense1_body,
        grid=(NRB,),
        in_specs=[
            pl.BlockSpec((G, ROWBLK, D), lambda i: (0, i, 0)),
            pl.BlockSpec((G, D, D), lambda i: (0, 0, 0)),
            pl.BlockSpec((1, D), lambda i: (0, 0)),
            pl.BlockSpec((1, 1, ROWBLK), lambda i: (i, 0, 0)),
        ],
        out_specs=[
            pl.BlockSpec((NUM_GRAPHS, D), lambda i: (0, 0)),
            pl.BlockSpec((NUM_GRAPHS, D), lambda i: (0, 0)),
        ],
        out_shape=[
            jax.ShapeDtypeStruct((NUM_GRAPHS, D), jnp.float32),
            jax.ShapeDtypeStruct((NUM_GRAPHS, D), jnp.float32),
        ],
    )(agg1.reshape(G, N, CW), Wop1, b_out1.reshape(1, D), batch3)

    return pl.pallas_call(
        _head_body,
        out_shape=jax.ShapeDtypeStruct((NUM_GRAPHS, OUT_DIM), jnp.float32),
    )(pool, cnt, fc1_W, fc1_b.reshape(1, OUT_DIM))


# speedup vs baseline: 1.3126x; 1.3126x over previous
import functools

import jax
import jax.numpy as jnp
import numpy as np
from jax import lax
from jax.experimental import pallas as pl
from jax.experimental.pallas import tpu as pltpu
from jax.experimental.pallas import tpu_sc as plsc

N = 10000
E = 160000
D = 128
HID = 4
G = 4
FS = 32
CW = FS * HID
ROW = 80
BLK = 5
EB = ROW * BLK
NT = 16
NROWS = E // ROW
RPT = NROWS // NT
NBLK = RPT // BLK
RNODE = N // NT
NUM_GRAPHS = 64
OUT_DIM = 10
ROWBLK = 1000
NRB = N // ROWBLK

_cc = np.arange(CW)
_PERM = np.stack([(32 * g + (_cc % 32)) * 4 + (_cc // 32) for g in range(G)])
_PERM_FLAT = _PERM.reshape(-1)


@functools.cache
def _get_rel_k():
    return functools.partial(
        pl.kernel,
        out_type=jax.ShapeDtypeStruct((E * 4,), jnp.float32),
        mesh=plsc.VectorSubcoreMesh(core_axis_name="c", subcore_axis_name="s"),
        scratch_types=[
            pltpu.VMEM((BLK, ROW), jnp.int32),
            pltpu.VMEM((BLK, ROW), jnp.int32),
            pltpu.VMEM((EB, 16), jnp.float32),
            pltpu.VMEM((EB, 16), jnp.float32),
            pltpu.VMEM((EB * 4,), jnp.float32),
            pltpu.SemaphoreType.DMA,
            pltpu.SemaphoreType.DMA,
        ],
        compiler_params=pltpu.CompilerParams(
            needs_layout_passes=False, use_tc_tiling_on_sc=False),
    )(_rel_body)


_NGRP = NROWS // BLK


def _rel_body(pos_hbm, src_hbm, dst_hbm, rel_hbm, sidx, didx, ps, pd, relv,
              sem_a, sem_g):
    c = lax.axis_index("c")
    s = lax.axis_index("s")
    w = s * 2 + c
    iota = lax.iota(jnp.int32, 16)

    def body(bi, _):
        grp = w + 32 * bi

        @pl.when(grp < _NGRP)
        def _():
            rowbase = grp * BLK
            ca1 = pltpu.async_copy(src_hbm.at[pl.ds(rowbase, BLK)], sidx, sem_a)
            ca2 = pltpu.async_copy(dst_hbm.at[pl.ds(rowbase, BLK)], didx, sem_a)
            ca1.wait()
            ca2.wait()
            cps = [pltpu.async_copy(pos_hbm.at[sidx.at[u]],
                                    ps.at[pl.ds(ROW * u, ROW)], sem_g)
                   for u in range(BLK)]
            cps += [pltpu.async_copy(pos_hbm.at[didx.at[u]],
                                     pd.at[pl.ds(ROW * u, ROW)], sem_g)
                    for u in range(BLK)]
            for cp in cps:
                cp.wait()
            i1 = iota % 4
            ibase = iota // 4

            def ext(ci, _):
                i0 = 4 * ci + ibase
                pv = plsc.load_gather(ps, [i0, i1])
                dv = plsc.load_gather(pd, [i0, i1])
                u = plsc.bitcast(pv - dv, jnp.int32)
                u = (u + 0x7FFF + (lax.shift_right_logical(u, 16) & 1)) & ~0xFFFF
                relv[pl.ds(16 * ci, 16)] = plsc.bitcast(u, jnp.float32)
                return 0

            lax.fori_loop(0, EB * 4 // 16, ext, 0)
            pltpu.sync_copy(relv, rel_hbm.at[pl.ds(rowbase * (ROW * 4), EB * 4)])

        return 0

    lax.fori_loop(0, (_NGRP + 31) // 32, body, 0)


@functools.cache
def _get_msg_k():
    return functools.partial(
        pl.kernel,
        out_type=jax.ShapeDtypeStruct((G * N, CW), jnp.float32),
        mesh=plsc.VectorSubcoreMesh(core_axis_name="c", subcore_axis_name="s"),
        scratch_types=[
            pltpu.VMEM_SHARED((N, CW), jnp.float32),
            pltpu.VMEM((2, ROW), jnp.int32),
            pltpu.VMEM((3, ROW), jnp.int32),
            pltpu.VMEM((2, ROW), jnp.int32),
            pltpu.VMEM((3, ROW * 4), jnp.float32),
            pltpu.VMEM((2, ROW, FS), jnp.float32),
            pltpu.VMEM((2, ROW, CW), jnp.float32),
            pltpu.VMEM((3, CW), jnp.float32),
            pltpu.VMEM((CW,), jnp.float32),
            pltpu.SemaphoreType.DMA((2,)),
            pltpu.SemaphoreType.DMA((3,)),
            pltpu.SemaphoreType.DMA((2,)),
            pltpu.SemaphoreType.DMA((2,)),
        ],
        compiler_params=pltpu.CompilerParams(
            needs_layout_passes=False, use_tc_tiling_on_sc=False),
    )(_msg_body)


def _msg_body(xflat_hbm, relf_hbm, src_hbm, dst_hbm, W_hbm, b_hbm, zeros_hbm,
              out_hbm, acc, sidx, didx, gidx, relv, xg, msg, Wg, bg,
              semi, semr, semg, sems):
    c = lax.axis_index("c")
    s = lax.axis_index("s")
    NSB = RPT

    def idx_loads(j, s2, s3):
        row = s * RPT + j
        ci = pltpu.async_copy(src_hbm.at[row], sidx.at[s2], semi.at[s2])
        cd = pltpu.async_copy(dst_hbm.at[row], didx.at[s3], semi.at[s2])
        cr = pltpu.async_copy(relf_hbm.at[pl.ds(row * (ROW * 4), ROW * 4)],
                              relv.at[s3], semr.at[s3])
        return ci, cd, cr

    def wait_idx(j, s2, s3):
        row = s * RPT + j
        pltpu.make_async_copy(src_hbm.at[row], sidx.at[s2], semi.at[s2]).wait()
        pltpu.make_async_copy(dst_hbm.at[row], didx.at[s3], semi.at[s2]).wait()

    for q in range(2):
        g = 2 * q + c
        pltpu.sync_copy(W_hbm.at[g], Wg)
        pltpu.sync_copy(b_hbm.at[g], bg)
        pltpu.sync_copy(zeros_hbm.at[pl.ds(s * RNODE, RNODE)],
                        acc.at[pl.ds(s * RNODE, RNODE)])
        plsc.subcore_barrier()

        gv = jnp.full((16,), g, dtype=jnp.int32)
        wv = [(Wg[0, pl.ds(16 * j, 16)], Wg[1, pl.ds(16 * j, 16)],
               Wg[2, pl.ds(16 * j, 16)], bg[pl.ds(16 * j, 16)])
              for j in range(CW // 16)]

        def prep_gather(j, s2):
            for i in range(ROW // 16):
                gidx[s2, pl.ds(16 * i, 16)] = (
                    lax.shift_left(sidx[s2, pl.ds(16 * i, 16)], 2) + gv)
            pltpu.async_copy(xflat_hbm.at[gidx.at[s2]], xg.at[s2], semg.at[s2])

        idx_loads(0, 0, 0)
        idx_loads(1, 1, 1)
        wait_idx(0, 0, 0)
        prep_gather(0, 0)

        def sub(j, _):
            sj2 = j & 1
            sj3 = lax.rem(j, 3)

            @pl.when(j + 1 < NSB)
            def _():
                s2 = (j + 1) & 1
                wait_idx(j + 1, s2, lax.rem(j + 1, 3))
                prep_gather(j + 1, s2)

            @pl.when(j + 2 < NSB)
            def _():
                idx_loads(j + 2, (j + 2) & 1, lax.rem(j + 2, 3))

            row = s * RPT + j
            pltpu.make_async_copy(xflat_hbm.at[gidx.at[sj2]], xg.at[sj2],
                                  semg.at[sj2]).wait()
            pltpu.make_async_copy(relf_hbm.at[pl.ds(row * (ROW * 4), ROW * 4)],
                                  relv.at[sj3], semr.at[sj3]).wait()

            def edge8(eg, _):
                rva = relv[sj3, pl.ds(32 * eg, 16)]
                rvb = relv[sj3, pl.ds(32 * eg + 16, 16)]
                for ie in range(8):
                    e = 8 * eg + ie
                    rv = rva if ie < 4 else rvb
                    q4 = (4 * ie) % 16
                    r0 = jnp.full((16,), rv[q4], dtype=jnp.float32)
                    r1 = jnp.full((16,), rv[q4 + 1], dtype=jnp.float32)
                    r2 = jnp.full((16,), rv[q4 + 2], dtype=jnp.float32)
                    xv0 = xg[sj2, e, pl.ds(0, 16)]
                    xv1 = xg[sj2, e, pl.ds(16, 16)]
                    for j8 in range(CW // 16):
                        w0, w1, w2, bb = wv[j8]
                        sp = jnp.maximum(w0 * r0 + w1 * r1 + w2 * r2 + bb, 0.0)
                        msg[sj2, e, pl.ds(16 * j8, 16)] = (
                            sp * (xv0 if j8 % 2 == 0 else xv1))
                return 0

            lax.fori_loop(0, ROW // 8, edge8, 0)

            @pl.when(j > 0)
            def _():
                sp2 = (j - 1) & 1
                pltpu.make_async_copy(
                    msg.at[sp2], acc.at[didx.at[lax.rem(j - 1, 3)]],
                    sems.at[sp2]).wait()

            pltpu.async_copy(msg.at[sj2], acc.at[didx.at[sj3]], sems.at[sj2],
                             add=True)
            return 0

        lax.fori_loop(0, NSB, sub, 0)
        pltpu.make_async_copy(msg.at[(NSB - 1) & 1],
                              acc.at[didx.at[(NSB - 1) % 3]],
                              sems.at[(NSB - 1) & 1]).wait()
        plsc.subcore_barrier()
        pltpu.sync_copy(acc.at[pl.ds(s * RNODE, RNODE)],
                        out_hbm.at[pl.ds(g * N + s * RNODE, RNODE)])
        plsc.subcore_barrier()


def _dense0_body(agg_ref, w_ref, b_ref, o_ref):
    h = jnp.dot(agg_ref[0], w_ref[0], preferred_element_type=jnp.float32)
    for g in range(1, G):
        h += jnp.dot(agg_ref[g], w_ref[g], preferred_element_type=jnp.float32)
    o_ref[...] = jnp.maximum(h + b_ref[...], 0.0)


def _dense1_body(agg_ref, w_ref, b_ref, batch_ref, pool_ref, cnt_ref):
    i = pl.program_id(0)

    @pl.when(i == 0)
    def _():
        pool_ref[...] = jnp.zeros_like(pool_ref)
        cnt_ref[...] = jnp.zeros_like(cnt_ref)

    h = jnp.dot(agg_ref[0], w_ref[0], preferred_element_type=jnp.float32)
    for g in range(1, G):
        h += jnp.dot(agg_ref[g], w_ref[g], preferred_element_type=jnp.float32)
    h = jnp.maximum(h + b_ref[...], 0.0)
    bt = batch_ref[0]
    ohT = (jnp.broadcast_to(bt, (NUM_GRAPHS, ROWBLK)) ==
           lax.broadcasted_iota(jnp.int32, (NUM_GRAPHS, ROWBLK), 0)
           ).astype(jnp.float32)
    pool_ref[...] += lax.dot_general(ohT, h, (((1,), (0,)), ((), ())),
                                     preferred_element_type=jnp.float32,
                                     precision=lax.Precision.HIGHEST)
    cnt_ref[...] += lax.dot_general(ohT, jnp.ones((ROWBLK, D), jnp.float32),
                                    (((1,), (0,)), ((), ())),
                                    preferred_element_type=jnp.float32, precision=lax.Precision.HIGHEST)


def _head_body(pool_ref, cnt_ref, w_ref, b_ref, o_ref):
    mean = pool_ref[...] / jnp.maximum(cnt_ref[...], 1.0)
    logits = jnp.dot(mean, w_ref[...], preferred_element_type=jnp.float32) + b_ref[...]
    m = jnp.max(logits, axis=1, keepdims=True)
    sh = logits - m
    o_ref[...] = sh - jnp.log(jnp.sum(jnp.exp(sh), axis=1, keepdims=True))


def _prep_w(W_in, b_in, W_out):
    W_in = W_in.astype(jnp.bfloat16).astype(jnp.float32)
    Wp = W_in[:, _PERM_FLAT].reshape(3, G, CW).transpose(1, 0, 2)
    bp = b_in[_PERM_FLAT].reshape(G, CW)
    Wop = W_out[_PERM_FLAT].reshape(G, CW, D)
    return Wp, bp, Wop


def kernel(x, pos, edge_index, batch, W_in0, b_in0, W_out0, b_out0,
           W_in1, b_in1, W_out1, b_out1, fc1_W, fc1_b):
    src2 = edge_index[0].reshape(NROWS, ROW).astype(jnp.int32)
    dst2 = edge_index[1].reshape(NROWS, ROW).astype(jnp.int32)
    pos4 = jnp.pad(pos, ((0, 0), (0, 13)))
    zeros = jnp.zeros((N, CW), jnp.float32)
    batch3 = batch.reshape(NRB, 1, ROWBLK).astype(jnp.int32)

    relf = _get_rel_k()(pos4, src2, dst2)

    Wp0, bp0, Wop0 = _prep_w(W_in0, b_in0, W_out0)
    Wp1, bp1, Wop1 = _prep_w(W_in1, b_in1, W_out1)

    xflat0 = x.reshape(G * N, FS)
    agg0 = _get_msg_k()(xflat0, relf, src2, dst2, Wp0, bp0, zeros)

    h1 = pl.pallas_call(
        _dense0_body,
        grid=(NRB,),
        in_specs=[
            pl.BlockSpec((G, ROWBLK, D), lambda i: (0, i, 0)),
            pl.BlockSpec((G, D, D), lambda i: (0, 0, 0)),
            pl.BlockSpec((1, D), lambda i: (0, 0)),
        ],
        out_specs=pl.BlockSpec((ROWBLK, D), lambda i: (i, 0)),
        out_shape=jax.ShapeDtypeStruct((N, D), jnp.float32),
    )(agg0.reshape(G, N, CW), Wop0, b_out0.reshape(1, D))

    agg1 = _get_msg_k()(h1.reshape(G * N, FS), relf, src2, dst2, Wp1, bp1, zeros)

    pool, cnt = pl.pallas_call(
        _dense1_body,
        grid=(NRB,),
        in_specs=[
            pl.BlockSpec((G, ROWBLK, D), lambda i: (0, i, 0)),
            pl.BlockSpec((G, D, D), lambda i: (0, 0, 0)),
            pl.BlockSpec((1, D), lambda i: (0, 0)),
            pl.BlockSpec((1, 1, ROWBLK), lambda i: (i, 0, 0)),
        ],
        out_specs=[
            pl.BlockSpec((NUM_GRAPHS, D), lambda i: (0, 0)),
            pl.BlockSpec((NUM_GRAPHS, D), lambda i: (0, 0)),
        ],
        out_shape=[
            jax.ShapeDtypeStruct((NUM_GRAPHS, D), jnp.float32),
            jax.ShapeDtypeStruct((NUM_GRAPHS, D), jnp.float32),
        ],
    )(agg1.reshape(G, N, CW), Wop1, b_out1.reshape(1, D), batch3)

    return pl.pallas_call(
        _head_body,
        out_shape=jax.ShapeDtypeStruct((NUM_GRAPHS, OUT_DIM), jnp.float32),
    )(pool, cnt, fc1_W, fc1_b.reshape(1, OUT_DIM))


# speedup vs baseline: 1.3148x; 1.0016x over previous
import functools

import jax
import jax.numpy as jnp
import numpy as np
from jax import lax
from jax.experimental import pallas as pl
from jax.experimental.pallas import tpu as pltpu
from jax.experimental.pallas import tpu_sc as plsc

N = 10000
E = 160000
D = 128
HID = 4
G = 4
FS = 32
CW = FS * HID
ROW = 80
BLK = 5
EB = ROW * BLK
NT = 16
NROWS = E // ROW
RPT = NROWS // NT
NBLK = RPT // BLK
RNODE = N // NT
NUM_GRAPHS = 64
OUT_DIM = 10
ROWBLK = 1000
NRB = N // ROWBLK

_cc = np.arange(CW)
_PERM = np.stack([(32 * g + (_cc % 32)) * 4 + (_cc // 32) for g in range(G)])
_PERM_FLAT = _PERM.reshape(-1)


@functools.cache
def _get_rel_k():
    return functools.partial(
        pl.kernel,
        out_type=jax.ShapeDtypeStruct((E * 4,), jnp.float32),
        mesh=plsc.VectorSubcoreMesh(core_axis_name="c", subcore_axis_name="s"),
        scratch_types=[
            pltpu.VMEM((BLK, ROW), jnp.int32),
            pltpu.VMEM((BLK, ROW), jnp.int32),
            pltpu.VMEM((EB, 16), jnp.float32),
            pltpu.VMEM((EB, 16), jnp.float32),
            pltpu.VMEM((EB * 4,), jnp.float32),
            pltpu.SemaphoreType.DMA,
            pltpu.SemaphoreType.DMA,
        ],
        compiler_params=pltpu.CompilerParams(
            needs_layout_passes=False, use_tc_tiling_on_sc=False),
    )(_rel_body)


_NGRP = NROWS // BLK


def _rel_body(pos_hbm, src_hbm, dst_hbm, rel_hbm, sidx, didx, ps, pd, relv,
              sem_a, sem_g):
    c = lax.axis_index("c")
    s = lax.axis_index("s")
    w = s * 2 + c
    iota = lax.iota(jnp.int32, 16)

    def body(bi, _):
        grp = w + 32 * bi

        @pl.when(grp < _NGRP)
        def _():
            rowbase = grp * BLK
            ca1 = pltpu.async_copy(src_hbm.at[pl.ds(rowbase, BLK)], sidx, sem_a)
            ca2 = pltpu.async_copy(dst_hbm.at[pl.ds(rowbase, BLK)], didx, sem_a)
            ca1.wait()
            ca2.wait()
            cps = [pltpu.async_copy(pos_hbm.at[sidx.at[u]],
                                    ps.at[pl.ds(ROW * u, ROW)], sem_g)
                   for u in range(BLK)]
            cps += [pltpu.async_copy(pos_hbm.at[didx.at[u]],
                                     pd.at[pl.ds(ROW * u, ROW)], sem_g)
                    for u in range(BLK)]
            for cp in cps:
                cp.wait()
            i1 = iota % 4
            ibase = iota // 4

            def ext(ci, _):
                i0 = 4 * ci + ibase
                pv = plsc.load_gather(ps, [i0, i1])
                dv = plsc.load_gather(pd, [i0, i1])
                u = plsc.bitcast(pv - dv, jnp.int32)
                u = (u + 0x7FFF + (lax.shift_right_logical(u, 16) & 1)) & ~0xFFFF
                relv[pl.ds(16 * ci, 16)] = plsc.bitcast(u, jnp.float32)
                return 0

            lax.fori_loop(0, EB * 4 // 16, ext, 0)
            pltpu.sync_copy(relv, rel_hbm.at[pl.ds(rowbase * (ROW * 4), EB * 4)])

        return 0

    lax.fori_loop(0, (_NGRP + 31) // 32, body, 0)


@functools.cache
def _get_msg_k():
    return functools.partial(
        pl.kernel,
        out_type=jax.ShapeDtypeStruct((G * N, CW), jnp.float32),
        mesh=plsc.VectorSubcoreMesh(core_axis_name="c", subcore_axis_name="s"),
        scratch_types=[
            pltpu.VMEM_SHARED((N, CW), jnp.float32),
            pltpu.VMEM((2, ROW), jnp.int32),
            pltpu.VMEM((3, ROW), jnp.int32),
            pltpu.VMEM((2, ROW), jnp.int32),
            pltpu.VMEM((3, ROW * 4), jnp.float32),
            pltpu.VMEM((2, ROW, FS), jnp.float32),
            pltpu.VMEM((2, ROW, CW), jnp.float32),
            pltpu.VMEM((3, CW), jnp.float32),
            pltpu.VMEM((CW,), jnp.float32),
            pltpu.SemaphoreType.DMA((2,)),
            pltpu.SemaphoreType.DMA((3,)),
            pltpu.SemaphoreType.DMA((2,)),
            pltpu.SemaphoreType.DMA((2,)),
        ],
        compiler_params=pltpu.CompilerParams(
            needs_layout_passes=False, use_tc_tiling_on_sc=False),
    )(_msg_body)


def _msg_body(xflat_hbm, relf_hbm, src_hbm, dst_hbm, W_hbm, b_hbm, zeros_hbm,
              out_hbm, acc, sidx, didx, gidx, relv, xg, msg, Wg, bg,
              semi, semr, semg, sems):
    c = lax.axis_index("c")
    s = lax.axis_index("s")
    NSB = RPT

    def idx_loads(j, s2, s3):
        row = s * RPT + j
        ci = pltpu.async_copy(src_hbm.at[row], sidx.at[s2], semi.at[s2])
        cd = pltpu.async_copy(dst_hbm.at[row], didx.at[s3], semi.at[s2])
        cr = pltpu.async_copy(relf_hbm.at[pl.ds(row * (ROW * 4), ROW * 4)],
                              relv.at[s3], semr.at[s3])
        return ci, cd, cr

    def wait_idx(j, s2, s3):
        row = s * RPT + j
        pltpu.make_async_copy(src_hbm.at[row], sidx.at[s2], semi.at[s2]).wait()
        pltpu.make_async_copy(dst_hbm.at[row], didx.at[s3], semi.at[s2]).wait()

    for q in range(2):
        g = 2 * q + c
        pltpu.sync_copy(W_hbm.at[g], Wg)
        pltpu.sync_copy(b_hbm.at[g], bg)
        pltpu.sync_copy(zeros_hbm.at[pl.ds(s * RNODE, RNODE)],
                        acc.at[pl.ds(s * RNODE, RNODE)])
        plsc.subcore_barrier()

        gv = jnp.full((16,), g, dtype=jnp.int32)
        wv = [(Wg[0, pl.ds(16 * j, 16)], Wg[1, pl.ds(16 * j, 16)],
               Wg[2, pl.ds(16 * j, 16)], bg[pl.ds(16 * j, 16)])
              for j in range(CW // 16)]

        def prep_gather(j, s2):
            for i in range(ROW // 16):
                gidx[s2, pl.ds(16 * i, 16)] = (
                    lax.shift_left(sidx[s2, pl.ds(16 * i, 16)], 2) + gv)
            pltpu.async_copy(xflat_hbm.at[gidx.at[s2]], xg.at[s2], semg.at[s2])

        idx_loads(0, 0, 0)
        idx_loads(1, 1, 1)
        wait_idx(0, 0, 0)
        prep_gather(0, 0)

        def sub(j, _):
            sj2 = j & 1
            sj3 = lax.rem(j, 3)

            @pl.when(j + 1 < NSB)
            def _():
                s2 = (j + 1) & 1
                wait_idx(j + 1, s2, lax.rem(j + 1, 3))
                prep_gather(j + 1, s2)

            @pl.when(j + 2 < NSB)
            def _():
                idx_loads(j + 2, (j + 2) & 1, lax.rem(j + 2, 3))

            row = s * RPT + j
            pltpu.make_async_copy(xflat_hbm.at[gidx.at[sj2]], xg.at[sj2],
                                  semg.at[sj2]).wait()
            pltpu.make_async_copy(relf_hbm.at[pl.ds(row * (ROW * 4), ROW * 4)],
                                  relv.at[sj3], semr.at[sj3]).wait()

            def edge8(eg, _):
                rva = relv[sj3, pl.ds(32 * eg, 16)]
                rvb = relv[sj3, pl.ds(32 * eg + 16, 16)]
                for ie in range(8):
                    e = 8 * eg + ie
                    rv = rva if ie < 4 else rvb
                    q4 = (4 * ie) % 16
                    r0 = jnp.full((16,), rv[q4], dtype=jnp.float32)
                    r1 = jnp.full((16,), rv[q4 + 1], dtype=jnp.float32)
                    r2 = jnp.full((16,), rv[q4 + 2], dtype=jnp.float32)
                    xv0 = xg[sj2, e, pl.ds(0, 16)]
                    xv1 = xg[sj2, e, pl.ds(16, 16)]
                    for j8 in range(CW // 16):
                        w0, w1, w2, bb = wv[j8]
                        sp = jnp.maximum(w0 * r0 + w1 * r1 + w2 * r2 + bb, 0.0)
                        msg[sj2, e, pl.ds(16 * j8, 16)] = (
                            sp * (xv0 if j8 % 2 == 0 else xv1))
                return 0

            lax.fori_loop(0, ROW // 8, edge8, 0)

            @pl.when(j > 0)
            def _():
                sp2 = (j - 1) & 1
                pltpu.make_async_copy(
                    msg.at[sp2], acc.at[didx.at[lax.rem(j - 1, 3)]],
                    sems.at[sp2]).wait()

            pltpu.async_copy(msg.at[sj2], acc.at[didx.at[sj3]], sems.at[sj2],
                             add=True)
            return 0

        lax.fori_loop(0, NSB, sub, 0)
        pltpu.make_async_copy(msg.at[(NSB - 1) & 1],
                              acc.at[didx.at[(NSB - 1) % 3]],
                              sems.at[(NSB - 1) & 1]).wait()
        plsc.subcore_barrier()
        pltpu.sync_copy(acc.at[pl.ds(s * RNODE, RNODE)],
                        out_hbm.at[pl.ds(g * N + s * RNODE, RNODE)])
        plsc.subcore_barrier()


def _dense0_body(agg_ref, w_ref, b_ref, o_ref):
    h = jnp.dot(agg_ref[0], w_ref[0], preferred_element_type=jnp.float32)
    for g in range(1, G):
        h += jnp.dot(agg_ref[g], w_ref[g], preferred_element_type=jnp.float32)
    o_ref[...] = jnp.maximum(h + b_ref[...], 0.0)


def _dense1_body(agg_ref, w_ref, b_ref, batch_ref, fw_ref, fb_ref, o_ref,
                 pool_ref, cnt_ref):
    i = pl.program_id(0)

    @pl.when(i == 0)
    def _():
        pool_ref[...] = jnp.zeros_like(pool_ref)
        cnt_ref[...] = jnp.zeros_like(cnt_ref)

    h = jnp.dot(agg_ref[0], w_ref[0], preferred_element_type=jnp.float32)
    for g in range(1, G):
        h += jnp.dot(agg_ref[g], w_ref[g], preferred_element_type=jnp.float32)
    h = jnp.maximum(h + b_ref[...], 0.0)
    bt = batch_ref[0]
    ohT = (jnp.broadcast_to(bt, (NUM_GRAPHS, ROWBLK)) ==
           lax.broadcasted_iota(jnp.int32, (NUM_GRAPHS, ROWBLK), 0)
           ).astype(jnp.float32)
    pool_ref[...] += lax.dot_general(ohT, h, (((1,), (0,)), ((), ())),
                                     preferred_element_type=jnp.float32,
                                     precision=lax.Precision.HIGHEST)
    cnt_ref[...] += lax.dot_general(ohT, jnp.ones((ROWBLK, D), jnp.float32),
                                    (((1,), (0,)), ((), ())),
                                    preferred_element_type=jnp.float32,
                                    precision=lax.Precision.HIGHEST)

    @pl.when(i == NRB - 1)
    def _():
        mean = pool_ref[...] / jnp.maximum(cnt_ref[...], 1.0)
        logits = jnp.dot(mean, fw_ref[...],
                         preferred_element_type=jnp.float32) + fb_ref[...]
        m = jnp.max(logits, axis=1, keepdims=True)
        sh = logits - m
        o_ref[...] = sh - jnp.log(jnp.sum(jnp.exp(sh), axis=1, keepdims=True))


def _prep_w(W_in, b_in, W_out):
    W_in = W_in.astype(jnp.bfloat16).astype(jnp.float32)
    Wp = W_in[:, _PERM_FLAT].reshape(3, G, CW).transpose(1, 0, 2)
    bp = b_in[_PERM_FLAT].reshape(G, CW)
    Wop = W_out[_PERM_FLAT].reshape(G, CW, D)
    return Wp, bp, Wop


def kernel(x, pos, edge_index, batch, W_in0, b_in0, W_out0, b_out0,
           W_in1, b_in1, W_out1, b_out1, fc1_W, fc1_b):
    src2 = edge_index[0].reshape(NROWS, ROW).astype(jnp.int32)
    dst2 = edge_index[1].reshape(NROWS, ROW).astype(jnp.int32)
    pos4 = jnp.pad(pos, ((0, 0), (0, 13)))
    zeros = jnp.zeros((N, CW), jnp.float32)
    batch3 = batch.reshape(NRB, 1, ROWBLK).astype(jnp.int32)

    relf = _get_rel_k()(pos4, src2, dst2)

    Wp0, bp0, Wop0 = _prep_w(W_in0, b_in0, W_out0)
    Wp1, bp1, Wop1 = _prep_w(W_in1, b_in1, W_out1)

    xflat0 = x.reshape(G * N, FS)
    agg0 = _get_msg_k()(xflat0, relf, src2, dst2, Wp0, bp0, zeros)

    h1 = pl.pallas_call(
        _dense0_body,
        grid=(NRB,),
        in_specs=[
            pl.BlockSpec((G, ROWBLK, D), lambda i: (0, i, 0)),
            pl.BlockSpec((G, D, D), lambda i: (0, 0, 0)),
            pl.BlockSpec((1, D), lambda i: (0, 0)),
        ],
        out_specs=pl.BlockSpec((ROWBLK, D), lambda i: (i, 0)),
        out_shape=jax.ShapeDtypeStruct((N, D), jnp.float32),
    )(agg0.reshape(G, N, CW), Wop0, b_out0.reshape(1, D))

    agg1 = _get_msg_k()(h1.reshape(G * N, FS), relf, src2, dst2, Wp1, bp1, zeros)

    return pl.pallas_call(
        _dense1_body,
        grid=(NRB,),
        in_specs=[
            pl.BlockSpec((G, ROWBLK, D), lambda i: (0, i, 0)),
            pl.BlockSpec((G, D, D), lambda i: (0, 0, 0)),
            pl.BlockSpec((1, D), lambda i: (0, 0)),
            pl.BlockSpec((1, 1, ROWBLK), lambda i: (i, 0, 0)),
            pl.BlockSpec((D, OUT_DIM), lambda i: (0, 0)),
            pl.BlockSpec((1, OUT_DIM), lambda i: (0, 0)),
        ],
        out_specs=pl.BlockSpec((NUM_GRAPHS, OUT_DIM), lambda i: (0, 0)),
        out_shape=jax.ShapeDtypeStruct((NUM_GRAPHS, OUT_DIM), jnp.float32),
        scratch_shapes=[
            pltpu.VMEM((NUM_GRAPHS, D), jnp.float32),
            pltpu.VMEM((NUM_GRAPHS, D), jnp.float32),
        ],
    )(agg1.reshape(G, N, CW), Wop1, b_out1.reshape(1, D), batch3,
      fc1_W, fc1_b.reshape(1, OUT_DIM))
